# Initial kernel scaffold; baseline (speedup 1.0000x reference)
import jax
import jax.numpy as jnp
from jax.experimental import pallas as pl


def kernel(x, edge_index, W1, al1, ar1, W2, al2, ar2):
    raise NotImplementedError("write your pallas kernel here")


# speedup vs baseline: 59.3604x; 59.3604x over previous
import functools

import jax
import jax.numpy as jnp
from jax import lax
from jax.experimental import pallas as pl
from jax.experimental.pallas import tpu as pltpu
from jax.experimental.pallas import tpu_sc as plsc

N = 10000
E = 320000
HEADS = 8
F = 16
FW = 144
NC, NS = 2, 16
NW = NC * NS
EPW = E // NW
C = 80
NCH = EPW // C
N_PAD = 10240
RPT = N_PAD // NS
BN = 1000


def _sc_gat_body(fsrc, edst, srci, dsti, out, src_idx, dst_idx, srows, drows,
                 msg, acc):
    cid = lax.axis_index("c")
    sid = lax.axis_index("s")
    wid = cid * NS + sid

    zero16 = jnp.zeros((16,), jnp.float32)

    def zrow(r, carry):
        for j in range(FW // 16):
            msg[r, pl.ds(j * 16, 16)] = zero16
        return carry

    lax.fori_loop(0, C, zrow, 0)
    for k in range(RPT // C):
        pltpu.sync_copy(msg, acc.at[pl.ds(sid * RPT + k * C, C)])
    plsc.subcore_barrier()

    def chunk_body(j, carry):
        pltpu.sync_copy(srci.at[wid, j], src_idx)
        pltpu.sync_copy(dsti.at[wid, j], dst_idx)
        pltpu.sync_copy(fsrc.at[src_idx], srows)
        pltpu.sync_copy(edst.at[dst_idx], drows)

        def edge_body(c, ecarry):
            el = srows[c, pl.ds(128, 16)]
            er = drows[c, pl.ds(0, 16)]
            s = el + er
            w16 = jnp.exp(jnp.where(s > 0, s, 0.2 * s))
            for h in range(HEADS):
                wh = w16[h]
                msg[c, pl.ds(h * 16, 16)] = srows[c, pl.ds(h * 16, 16)] * wh
            msg[c, pl.ds(128, 16)] = w16
            return ecarry

        lax.fori_loop(0, C, edge_body, 0)
        pltpu.sync_copy(msg, acc.at[dst_idx], add=True)
        return carry

    lax.fori_loop(0, NCH, chunk_body, 0)
    plsc.subcore_barrier()

    for k in range(RPT // C):
        r0 = sid * RPT + k * C
        pltpu.sync_copy(acc.at[pl.ds(r0, C)], out.at[cid, pl.ds(r0, C)])


_sc_gat = pl.kernel(
    _sc_gat_body,
    mesh=plsc.VectorSubcoreMesh(core_axis_name="c", subcore_axis_name="s"),
    compiler_params=pltpu.CompilerParams(use_tc_tiling_on_sc=False),
    out_type=jax.ShapeDtypeStruct((NC, N_PAD, FW), jnp.float32),
    scratch_types=[
        pltpu.VMEM((C,), jnp.int32),
        pltpu.VMEM((C,), jnp.int32),
        pltpu.VMEM((C, FW), jnp.float32),
        pltpu.VMEM((C, 16), jnp.float32),
        pltpu.VMEM((C, FW), jnp.float32),
        pltpu.VMEM_SHARED((N_PAD, FW), jnp.float32),
    ],
)


def _pre_body(x_ref, w_ref, a_ref, b_ref, fsrc_ref, edst_ref):
    feat = jnp.dot(x_ref[...], w_ref[...], preferred_element_type=jnp.float32)
    el = jnp.dot(feat, a_ref[...], preferred_element_type=jnp.float32)
    er = jnp.dot(feat, b_ref[...], preferred_element_type=jnp.float32)
    z8 = jnp.zeros((feat.shape[0], 8), jnp.float32)
    fsrc_ref[...] = jnp.concatenate([feat, el, z8], axis=1)
    edst_ref[...] = jnp.concatenate([er, z8], axis=1)


def _divide_heads(s):
    cols = []
    for h in range(HEADS):
        den = s[:, 128 + h:129 + h] + 1e-9
        cols.append(s[:, h * 16:(h + 1) * 16] / den)
    return jnp.concatenate(cols, axis=1)


def _mid_body(p_ref, w_ref, a_ref, b_ref, fsrc_ref, edst_ref):
    p = p_ref[...]
    s = p[0] + p[1]
    o = _divide_heads(s)
    hmean = jnp.zeros((o.shape[0], F), jnp.float32)
    for h in range(HEADS):
        hmean = hmean + o[:, h * 16:(h + 1) * 16]
    hmean = hmean * (1.0 / HEADS)
    helu = jnp.where(hmean > 0, hmean, jnp.exp(hmean) - 1.0)
    feat = jnp.dot(helu, w_ref[...], preferred_element_type=jnp.float32)
    el = jnp.dot(feat, a_ref[...], preferred_element_type=jnp.float32)
    er = jnp.dot(feat, b_ref[...], preferred_element_type=jnp.float32)
    z8 = jnp.zeros((feat.shape[0], 8), jnp.float32)
    fsrc_ref[...] = jnp.concatenate([feat, el, z8], axis=1)
    edst_ref[...] = jnp.concatenate([er, z8], axis=1)


def _post_body(p_ref, out_ref):
    p = p_ref[...]
    out_ref[...] = _divide_heads(p[0] + p[1])


_GRID = N // BN

_pre_call = pl.pallas_call(
    _pre_body,
    grid=(_GRID,),
    in_specs=[
        pl.BlockSpec((BN, 128), lambda i: (i, 0)),
        pl.BlockSpec((128, HEADS * F), lambda i: (0, 0)),
        pl.BlockSpec((HEADS * F, HEADS), lambda i: (0, 0)),
        pl.BlockSpec((HEADS * F, HEADS), lambda i: (0, 0)),
    ],
    out_specs=[
        pl.BlockSpec((BN, FW), lambda i: (i, 0)),
        pl.BlockSpec((BN, 16), lambda i: (i, 0)),
    ],
    out_shape=[
        jax.ShapeDtypeStruct((N, FW), jnp.float32),
        jax.ShapeDtypeStruct((N, 16), jnp.float32),
    ],
)

_mid_call = pl.pallas_call(
    _mid_body,
    grid=(_GRID,),
    in_specs=[
        pl.BlockSpec((NC, BN, FW), lambda i: (0, i, 0)),
        pl.BlockSpec((F, HEADS * F), lambda i: (0, 0)),
        pl.BlockSpec((HEADS * F, HEADS), lambda i: (0, 0)),
        pl.BlockSpec((HEADS * F, HEADS), lambda i: (0, 0)),
    ],
    out_specs=[
        pl.BlockSpec((BN, FW), lambda i: (i, 0)),
        pl.BlockSpec((BN, 16), lambda i: (i, 0)),
    ],
    out_shape=[
        jax.ShapeDtypeStruct((N, FW), jnp.float32),
        jax.ShapeDtypeStruct((N, 16), jnp.float32),
    ],
)

_post_call = pl.pallas_call(
    _post_body,
    grid=(_GRID,),
    in_specs=[pl.BlockSpec((NC, BN, FW), lambda i: (0, i, 0))],
    out_specs=pl.BlockSpec((BN, HEADS * F), lambda i: (i, 0)),
    out_shape=jax.ShapeDtypeStruct((N, HEADS * F), jnp.float32),
)


def _attn_mat(a):
    hf = HEADS * F
    rows = jnp.arange(hf)
    return jnp.zeros((hf, HEADS), jnp.float32).at[rows, rows // F].set(
        a.reshape(-1))


def kernel(x, edge_index, W1, al1, ar1, W2, al2, ar2):
    src = edge_index[0].reshape(NW, NCH, C)
    dst = edge_index[1].reshape(NW, NCH, C)
    A1, B1 = _attn_mat(al1), _attn_mat(ar1)
    A2, B2 = _attn_mat(al2), _attn_mat(ar2)

    fsrc1, edst1 = _pre_call(x, W1, A1, B1)
    p1 = _sc_gat(fsrc1, edst1, src, dst)
    fsrc2, edst2 = _mid_call(p1, W2, A2, B2)
    p2 = _sc_gat(fsrc2, edst2, src, dst)
    out = _post_call(p2)
    return out.reshape(N, HEADS, F)


# speedup vs baseline: 94.7660x; 1.5965x over previous
import functools

import jax
import jax.numpy as jnp
from jax import lax
from jax.experimental import pallas as pl
from jax.experimental.pallas import tpu as pltpu
from jax.experimental.pallas import tpu_sc as plsc

N = 10000
E = 320000
HEADS = 8
F = 16
FW = 144
NC, NS = 2, 16
NW = NC * NS
EPW = E // NW
C = 80
NCH = EPW // C
N_PAD = 10240
RPT = N_PAD // NS
BN = 1000


def _sc_gat_body(fsrc, edst, eidx, out,
                 eidx0, eidx1, srows0, srows1, drows0, drows1,
                 gsem0, gsem1, ssem0, ssem1, acc):
    cid = lax.axis_index("c")
    sid = lax.axis_index("s")
    wid = cid * NS + sid
    buf0 = (eidx0, srows0, drows0, gsem0, ssem0)
    buf1 = (eidx1, srows1, drows1, gsem1, ssem1)

    zero16 = jnp.zeros((16,), jnp.float32)

    def zrow(r, carry):
        for j in range(FW // 16):
            srows0[r, pl.ds(j * 16, 16)] = zero16
        return carry

    lax.fori_loop(0, C, zrow, 0)
    for k in range(RPT // C):
        pltpu.sync_copy(srows0, acc.at[pl.ds(sid * RPT + k * C, C)])
    plsc.subcore_barrier()

    def step(j, cur, nxt, first, last):
        ceidx, csr, cdr, cgsem, cssem = cur
        neidx, nsr, ndr, ngsem, nssem = nxt
        pltpu.make_async_copy(fsrc.at[ceidx.at[0]], csr, cgsem).wait()
        pltpu.make_async_copy(edst.at[ceidx.at[1]], cdr, cgsem).wait()
        if not first:
            pltpu.make_async_copy(nsr, acc.at[neidx.at[1]], nssem).wait()
        if not last:
            pltpu.sync_copy(eidx.at[wid, j + 1], neidx)
            pltpu.async_copy(fsrc.at[neidx.at[0]], nsr, ngsem)
            pltpu.async_copy(edst.at[neidx.at[1]], ndr, ngsem)

        def edge_body(c, ecarry):
            el = csr[c, pl.ds(128, 16)]
            er = cdr[c, pl.ds(0, 16)]
            s = el + er
            w16 = jnp.exp(jnp.where(s > 0, s, 0.2 * s))
            for h in range(HEADS):
                wh = w16[h]
                csr[c, pl.ds(h * 16, 16)] = csr[c, pl.ds(h * 16, 16)] * wh
            csr[c, pl.ds(128, 16)] = w16
            return ecarry

        lax.fori_loop(0, C, edge_body, 0, unroll=2)
        pltpu.async_copy(csr, acc.at[ceidx.at[1]], cssem, add=True)

    pltpu.sync_copy(eidx.at[wid, 0], eidx0)
    pltpu.async_copy(fsrc.at[eidx0.at[0]], srows0, gsem0)
    pltpu.async_copy(edst.at[eidx0.at[1]], drows0, gsem0)

    step(0, buf0, buf1, True, False)
    step(1, buf1, buf0, False, False)

    def pair(t, carry):
        step(2 * t, buf0, buf1, False, False)
        step(2 * t + 1, buf1, buf0, False, False)
        return carry

    lax.fori_loop(1, NCH // 2, pair, 0)
    step(NCH - 1, buf0, buf1, False, True)
    pltpu.make_async_copy(srows0, acc.at[eidx0.at[1]], ssem0).wait()
    plsc.subcore_barrier()

    for k in range(RPT // C):
        r0 = sid * RPT + k * C
        pltpu.sync_copy(acc.at[pl.ds(r0, C)], out.at[cid, pl.ds(r0, C)])


_sc_gat = pl.kernel(
    _sc_gat_body,
    mesh=plsc.VectorSubcoreMesh(core_axis_name="c", subcore_axis_name="s"),
    compiler_params=pltpu.CompilerParams(use_tc_tiling_on_sc=False),
    out_type=jax.ShapeDtypeStruct((NC, N_PAD, FW), jnp.float32),
    scratch_types=[
        pltpu.VMEM((2, C), jnp.int32),
        pltpu.VMEM((2, C), jnp.int32),
        pltpu.VMEM((C, FW), jnp.float32),
        pltpu.VMEM((C, FW), jnp.float32),
        pltpu.VMEM((C, 16), jnp.float32),
        pltpu.VMEM((C, 16), jnp.float32),
        pltpu.SemaphoreType.DMA,
        pltpu.SemaphoreType.DMA,
        pltpu.SemaphoreType.DMA,
        pltpu.SemaphoreType.DMA,
        pltpu.VMEM_SHARED((N_PAD, FW), jnp.float32),
    ],
)


def _pre_body(x_ref, w_ref, a_ref, b_ref, fsrc_ref, edst_ref):
    feat = jnp.dot(x_ref[...], w_ref[...], preferred_element_type=jnp.float32)
    el = jnp.dot(feat, a_ref[...], preferred_element_type=jnp.float32)
    er = jnp.dot(feat, b_ref[...], preferred_element_type=jnp.float32)
    z8 = jnp.zeros((feat.shape[0], 8), jnp.float32)
    fsrc_ref[...] = jnp.concatenate([feat, el, z8], axis=1)
    edst_ref[...] = jnp.concatenate([er, z8], axis=1)


def _divide_heads(s):
    cols = []
    for h in range(HEADS):
        den = s[:, 128 + h:129 + h] + 1e-9
        cols.append(s[:, h * 16:(h + 1) * 16] / den)
    return jnp.concatenate(cols, axis=1)


def _mid_body(p_ref, w_ref, a_ref, b_ref, fsrc_ref, edst_ref):
    p = p_ref[...]
    s = p[0] + p[1]
    o = _divide_heads(s)
    hmean = jnp.zeros((o.shape[0], F), jnp.float32)
    for h in range(HEADS):
        hmean = hmean + o[:, h * 16:(h + 1) * 16]
    hmean = hmean * (1.0 / HEADS)
    helu = jnp.where(hmean > 0, hmean, jnp.exp(hmean) - 1.0)
    feat = jnp.dot(helu, w_ref[...], preferred_element_type=jnp.float32)
    el = jnp.dot(feat, a_ref[...], preferred_element_type=jnp.float32)
    er = jnp.dot(feat, b_ref[...], preferred_element_type=jnp.float32)
    z8 = jnp.zeros((feat.shape[0], 8), jnp.float32)
    fsrc_ref[...] = jnp.concatenate([feat, el, z8], axis=1)
    edst_ref[...] = jnp.concatenate([er, z8], axis=1)


def _post_body(p_ref, out_ref):
    p = p_ref[...]
    out_ref[...] = _divide_heads(p[0] + p[1])


_GRID = N // BN

_pre_call = pl.pallas_call(
    _pre_body,
    grid=(_GRID,),
    in_specs=[
        pl.BlockSpec((BN, 128), lambda i: (i, 0)),
        pl.BlockSpec((128, HEADS * F), lambda i: (0, 0)),
        pl.BlockSpec((HEADS * F, HEADS), lambda i: (0, 0)),
        pl.BlockSpec((HEADS * F, HEADS), lambda i: (0, 0)),
    ],
    out_specs=[
        pl.BlockSpec((BN, FW), lambda i: (i, 0)),
        pl.BlockSpec((BN, 16), lambda i: (i, 0)),
    ],
    out_shape=[
        jax.ShapeDtypeStruct((N, FW), jnp.float32),
        jax.ShapeDtypeStruct((N, 16), jnp.float32),
    ],
)

_mid_call = pl.pallas_call(
    _mid_body,
    grid=(_GRID,),
    in_specs=[
        pl.BlockSpec((NC, BN, FW), lambda i: (0, i, 0)),
        pl.BlockSpec((F, HEADS * F), lambda i: (0, 0)),
        pl.BlockSpec((HEADS * F, HEADS), lambda i: (0, 0)),
        pl.BlockSpec((HEADS * F, HEADS), lambda i: (0, 0)),
    ],
    out_specs=[
        pl.BlockSpec((BN, FW), lambda i: (i, 0)),
        pl.BlockSpec((BN, 16), lambda i: (i, 0)),
    ],
    out_shape=[
        jax.ShapeDtypeStruct((N, FW), jnp.float32),
        jax.ShapeDtypeStruct((N, 16), jnp.float32),
    ],
)

_post_call = pl.pallas_call(
    _post_body,
    grid=(_GRID,),
    in_specs=[pl.BlockSpec((NC, BN, FW), lambda i: (0, i, 0))],
    out_specs=pl.BlockSpec((BN, HEADS * F), lambda i: (i, 0)),
    out_shape=jax.ShapeDtypeStruct((N, HEADS * F), jnp.float32),
)


def _attn_mat(a):
    hf = HEADS * F
    rows = jnp.arange(hf)
    return jnp.zeros((hf, HEADS), jnp.float32).at[rows, rows // F].set(
        a.reshape(-1))


def kernel(x, edge_index, W1, al1, ar1, W2, al2, ar2):
    eidx = jnp.stack([edge_index[0].reshape(NW, NCH, C),
                      edge_index[1].reshape(NW, NCH, C)], axis=2)
    A1, B1 = _attn_mat(al1), _attn_mat(ar1)
    A2, B2 = _attn_mat(al2), _attn_mat(ar2)

    fsrc1, edst1 = _pre_call(x, W1, A1, B1)
    p1 = _sc_gat(fsrc1, edst1, eidx)
    fsrc2, edst2 = _mid_call(p1, W2, A2, B2)
    p2 = _sc_gat(fsrc2, edst2, eidx)
    out = _post_call(p2)
    return out.reshape(N, HEADS, F)


# speedup vs baseline: 102.8950x; 1.0858x over previous
import functools

import jax
import jax.numpy as jnp
from jax import lax
from jax.experimental import pallas as pl
from jax.experimental.pallas import tpu as pltpu
from jax.experimental.pallas import tpu_sc as plsc

N = 10000
E = 320000
HEADS = 8
F = 16
FW = 144
NC, NS = 2, 16
NW = NC * NS
EPW = E // NW
C = 80
NCH = EPW // C
N_PAD = 10240
RPT = N_PAD // NS
BN = 1000


def _sc_gat_body(fsrc, edst, eidx, out,
                 eidx0, eidx1, srows0, srows1, drows0, drows1,
                 gsem0, gsem1, ssem0, ssem1, acc):
    cid = lax.axis_index("c")
    sid = lax.axis_index("s")
    wid = cid * NS + sid
    buf0 = (eidx0, srows0, drows0, gsem0, ssem0)
    buf1 = (eidx1, srows1, drows1, gsem1, ssem1)

    zero16 = jnp.zeros((16,), jnp.float32)

    def zrow(r, carry):
        for j in range(FW // 16):
            srows0[r, pl.ds(j * 16, 16)] = zero16
        return carry

    lax.fori_loop(0, C, zrow, 0)
    for k in range(RPT // C):
        pltpu.sync_copy(srows0, acc.at[pl.ds(sid * RPT + k * C, C)])
    plsc.subcore_barrier()

    def step(j, cur, nxt, first, last):
        ceidx, csr, cdr, cgsem, cssem = cur
        neidx, nsr, ndr, ngsem, nssem = nxt
        pltpu.make_async_copy(fsrc.at[ceidx.at[0]], csr, cgsem).wait()
        pltpu.make_async_copy(edst.at[ceidx.at[1]], cdr, cgsem).wait()
        if not first:
            pltpu.make_async_copy(nsr, acc.at[neidx.at[1]], nssem).wait()
        if not last:
            pltpu.async_copy(eidx.at[wid, j + 1], neidx, ngsem)

        def edge_body(c, ecarry):
            el = csr[c, pl.ds(128, 16)]
            er = cdr[c, pl.ds(0, 16)]
            s = el + er
            w16 = jnp.exp(jnp.where(s > 0, s, 0.2 * s))
            for h in range(HEADS):
                wh = w16[h]
                csr[c, pl.ds(h * 16, 16)] = csr[c, pl.ds(h * 16, 16)] * wh
            csr[c, pl.ds(128, 16)] = w16
            return ecarry

        PRE = 16
        lax.fori_loop(0, PRE, edge_body, 0, unroll=4)
        if not last:
            pltpu.make_async_copy(eidx.at[wid, j + 1], neidx, ngsem).wait()
            pltpu.async_copy(fsrc.at[neidx.at[0]], nsr, ngsem)
            pltpu.async_copy(edst.at[neidx.at[1]], ndr, ngsem)
        lax.fori_loop(PRE, C, edge_body, 0, unroll=4)
        pltpu.async_copy(csr, acc.at[ceidx.at[1]], cssem, add=True)

    pltpu.sync_copy(eidx.at[wid, 0], eidx0)
    pltpu.async_copy(fsrc.at[eidx0.at[0]], srows0, gsem0)
    pltpu.async_copy(edst.at[eidx0.at[1]], drows0, gsem0)

    step(0, buf0, buf1, True, False)
    step(1, buf1, buf0, False, False)

    def pair(t, carry):
        step(2 * t, buf0, buf1, False, False)
        step(2 * t + 1, buf1, buf0, False, False)
        return carry

    lax.fori_loop(1, NCH // 2, pair, 0)
    step(NCH - 1, buf0, buf1, False, True)
    pltpu.make_async_copy(srows0, acc.at[eidx0.at[1]], ssem0).wait()
    plsc.subcore_barrier()

    for k in range(RPT // C):
        r0 = sid * RPT + k * C
        pltpu.sync_copy(acc.at[pl.ds(r0, C)], out.at[cid, pl.ds(r0, C)])


_sc_gat = pl.kernel(
    _sc_gat_body,
    mesh=plsc.VectorSubcoreMesh(core_axis_name="c", subcore_axis_name="s"),
    compiler_params=pltpu.CompilerParams(use_tc_tiling_on_sc=False),
    out_type=jax.ShapeDtypeStruct((NC, N_PAD, FW), jnp.float32),
    scratch_types=[
        pltpu.VMEM((2, C), jnp.int32),
        pltpu.VMEM((2, C), jnp.int32),
        pltpu.VMEM((C, FW), jnp.float32),
        pltpu.VMEM((C, FW), jnp.float32),
        pltpu.VMEM((C, 16), jnp.float32),
        pltpu.VMEM((C, 16), jnp.float32),
        pltpu.SemaphoreType.DMA,
        pltpu.SemaphoreType.DMA,
        pltpu.SemaphoreType.DMA,
        pltpu.SemaphoreType.DMA,
        pltpu.VMEM_SHARED((N_PAD, FW), jnp.float32),
    ],
)


def _pre_body(x_ref, w_ref, a_ref, b_ref, fsrc_ref, edst_ref):
    feat = jnp.dot(x_ref[...], w_ref[...], preferred_element_type=jnp.float32)
    el = jnp.dot(feat, a_ref[...], preferred_element_type=jnp.float32)
    er = jnp.dot(feat, b_ref[...], preferred_element_type=jnp.float32)
    z8 = jnp.zeros((feat.shape[0], 8), jnp.float32)
    fsrc_ref[...] = jnp.concatenate([feat, el, z8], axis=1)
    edst_ref[...] = jnp.concatenate([er, z8], axis=1)


def _divide_heads(s):
    cols = []
    for h in range(HEADS):
        den = s[:, 128 + h:129 + h] + 1e-9
        cols.append(s[:, h * 16:(h + 1) * 16] / den)
    return jnp.concatenate(cols, axis=1)


def _mid_body(p_ref, w_ref, a_ref, b_ref, fsrc_ref, edst_ref):
    p = p_ref[...]
    s = p[0] + p[1]
    o = _divide_heads(s)
    hmean = jnp.zeros((o.shape[0], F), jnp.float32)
    for h in range(HEADS):
        hmean = hmean + o[:, h * 16:(h + 1) * 16]
    hmean = hmean * (1.0 / HEADS)
    helu = jnp.where(hmean > 0, hmean, jnp.exp(hmean) - 1.0)
    feat = jnp.dot(helu, w_ref[...], preferred_element_type=jnp.float32)
    el = jnp.dot(feat, a_ref[...], preferred_element_type=jnp.float32)
    er = jnp.dot(feat, b_ref[...], preferred_element_type=jnp.float32)
    z8 = jnp.zeros((feat.shape[0], 8), jnp.float32)
    fsrc_ref[...] = jnp.concatenate([feat, el, z8], axis=1)
    edst_ref[...] = jnp.concatenate([er, z8], axis=1)


def _post_body(p_ref, out_ref):
    p = p_ref[...]
    out_ref[...] = _divide_heads(p[0] + p[1])


_GRID = N // BN

_pre_call = pl.pallas_call(
    _pre_body,
    grid=(_GRID,),
    in_specs=[
        pl.BlockSpec((BN, 128), lambda i: (i, 0)),
        pl.BlockSpec((128, HEADS * F), lambda i: (0, 0)),
        pl.BlockSpec((HEADS * F, HEADS), lambda i: (0, 0)),
        pl.BlockSpec((HEADS * F, HEADS), lambda i: (0, 0)),
    ],
    out_specs=[
        pl.BlockSpec((BN, FW), lambda i: (i, 0)),
        pl.BlockSpec((BN, 16), lambda i: (i, 0)),
    ],
    out_shape=[
        jax.ShapeDtypeStruct((N, FW), jnp.float32),
        jax.ShapeDtypeStruct((N, 16), jnp.float32),
    ],
)

_mid_call = pl.pallas_call(
    _mid_body,
    grid=(_GRID,),
    in_specs=[
        pl.BlockSpec((NC, BN, FW), lambda i: (0, i, 0)),
        pl.BlockSpec((F, HEADS * F), lambda i: (0, 0)),
        pl.BlockSpec((HEADS * F, HEADS), lambda i: (0, 0)),
        pl.BlockSpec((HEADS * F, HEADS), lambda i: (0, 0)),
    ],
    out_specs=[
        pl.BlockSpec((BN, FW), lambda i: (i, 0)),
        pl.BlockSpec((BN, 16), lambda i: (i, 0)),
    ],
    out_shape=[
        jax.ShapeDtypeStruct((N, FW), jnp.float32),
        jax.ShapeDtypeStruct((N, 16), jnp.float32),
    ],
)

_post_call = pl.pallas_call(
    _post_body,
    grid=(_GRID,),
    in_specs=[pl.BlockSpec((NC, BN, FW), lambda i: (0, i, 0))],
    out_specs=pl.BlockSpec((BN, HEADS * F), lambda i: (i, 0)),
    out_shape=jax.ShapeDtypeStruct((N, HEADS * F), jnp.float32),
)


def _attn_mat(a):
    hf = HEADS * F
    rows = jnp.arange(hf)
    return jnp.zeros((hf, HEADS), jnp.float32).at[rows, rows // F].set(
        a.reshape(-1))


def kernel(x, edge_index, W1, al1, ar1, W2, al2, ar2):
    eidx = jnp.stack([edge_index[0].reshape(NW, NCH, C),
                      edge_index[1].reshape(NW, NCH, C)], axis=2)
    A1, B1 = _attn_mat(al1), _attn_mat(ar1)
    A2, B2 = _attn_mat(al2), _attn_mat(ar2)

    fsrc1, edst1 = _pre_call(x, W1, A1, B1)
    p1 = _sc_gat(fsrc1, edst1, eidx)
    fsrc2, edst2 = _mid_call(p1, W2, A2, B2)
    p2 = _sc_gat(fsrc2, edst2, eidx)
    out = _post_call(p2)
    return out.reshape(N, HEADS, F)


# speedup vs baseline: 104.4429x; 1.0150x over previous
import functools

import jax
import jax.numpy as jnp
from jax import lax
from jax.experimental import pallas as pl
from jax.experimental.pallas import tpu as pltpu
from jax.experimental.pallas import tpu_sc as plsc

N = 10000
E = 320000
HEADS = 8
F = 16
FW = 144
NC, NS = 2, 16
NW = NC * NS
EPW = E // NW
C = 100
NCH = EPW // C
N_PAD = 10240
RPT = N_PAD // NS
ZR = 80
BN = 1000


def _sc_gat_body(fsrc, edst, eidx, out,
                 eidx0, eidx1, srows0, srows1, drows0, drows1,
                 gsem0, gsem1, ssem0, ssem1, acc):
    cid = lax.axis_index("c")
    sid = lax.axis_index("s")
    wid = cid * NS + sid
    buf0 = (eidx0, srows0, drows0, gsem0, ssem0)
    buf1 = (eidx1, srows1, drows1, gsem1, ssem1)

    zero16 = jnp.zeros((16,), jnp.float32)

    def zrow(r, carry):
        for j in range(FW // 16):
            srows0[r, pl.ds(j * 16, 16)] = zero16
        return carry

    lax.fori_loop(0, ZR, zrow, 0)
    for k in range(RPT // ZR):
        pltpu.sync_copy(srows0.at[pl.ds(0, ZR)],
                        acc.at[pl.ds(sid * RPT + k * ZR, ZR)])
    plsc.subcore_barrier()

    def step(j, cur, nxt, first, last):
        ceidx, csr, cdr, cgsem, cssem = cur
        neidx, nsr, ndr, ngsem, nssem = nxt
        pltpu.make_async_copy(fsrc.at[ceidx.at[0]], csr, cgsem).wait()
        pltpu.make_async_copy(edst.at[ceidx.at[1]], cdr, cgsem).wait()
        if not first:
            pltpu.make_async_copy(nsr, acc.at[neidx.at[1]], nssem).wait()
        if not last:
            pltpu.async_copy(eidx.at[wid, j + 1], neidx, ngsem)

        def edge_body(c, ecarry):
            el = csr[c, pl.ds(128, 16)]
            er = cdr[c, pl.ds(0, 16)]
            s = el + er
            w16 = jnp.exp(jnp.where(s > 0, s, 0.2 * s))
            for h in range(HEADS):
                wh = w16[h]
                csr[c, pl.ds(h * 16, 16)] = csr[c, pl.ds(h * 16, 16)] * wh
            csr[c, pl.ds(128, 16)] = w16
            return ecarry

        PRE = 16
        lax.fori_loop(0, PRE, edge_body, 0, unroll=4)
        if not last:
            pltpu.make_async_copy(eidx.at[wid, j + 1], neidx, ngsem).wait()
            pltpu.async_copy(fsrc.at[neidx.at[0]], nsr, ngsem)
            pltpu.async_copy(edst.at[neidx.at[1]], ndr, ngsem)
        lax.fori_loop(PRE, C, edge_body, 0, unroll=4)
        pltpu.async_copy(csr, acc.at[ceidx.at[1]], cssem, add=True)

    pltpu.sync_copy(eidx.at[wid, 0], eidx0)
    pltpu.async_copy(fsrc.at[eidx0.at[0]], srows0, gsem0)
    pltpu.async_copy(edst.at[eidx0.at[1]], drows0, gsem0)

    step(0, buf0, buf1, True, False)
    step(1, buf1, buf0, False, False)

    def pair(t, carry):
        step(2 * t, buf0, buf1, False, False)
        step(2 * t + 1, buf1, buf0, False, False)
        return carry

    lax.fori_loop(1, NCH // 2 - 1, pair, 0)
    step(NCH - 2, buf0, buf1, False, False)
    step(NCH - 1, buf1, buf0, False, True)
    pltpu.make_async_copy(srows1, acc.at[eidx1.at[1]], ssem1).wait()
    plsc.subcore_barrier()

    for k in range(RPT // ZR):
        r0 = sid * RPT + k * ZR
        pltpu.sync_copy(acc.at[pl.ds(r0, ZR)], out.at[cid, pl.ds(r0, ZR)])


_sc_gat = pl.kernel(
    _sc_gat_body,
    mesh=plsc.VectorSubcoreMesh(core_axis_name="c", subcore_axis_name="s"),
    compiler_params=pltpu.CompilerParams(use_tc_tiling_on_sc=False),
    out_type=jax.ShapeDtypeStruct((NC, N_PAD, FW), jnp.float32),
    scratch_types=[
        pltpu.VMEM((2, C), jnp.int32),
        pltpu.VMEM((2, C), jnp.int32),
        pltpu.VMEM((C, FW), jnp.float32),
        pltpu.VMEM((C, FW), jnp.float32),
        pltpu.VMEM((C, 16), jnp.float32),
        pltpu.VMEM((C, 16), jnp.float32),
        pltpu.SemaphoreType.DMA,
        pltpu.SemaphoreType.DMA,
        pltpu.SemaphoreType.DMA,
        pltpu.SemaphoreType.DMA,
        pltpu.VMEM_SHARED((N_PAD, FW), jnp.float32),
    ],
)


def _pre_body(x_ref, w_ref, a_ref, b_ref, fsrc_ref, edst_ref):
    feat = jnp.dot(x_ref[...], w_ref[...], preferred_element_type=jnp.float32)
    el = jnp.dot(feat, a_ref[...], preferred_element_type=jnp.float32)
    er = jnp.dot(feat, b_ref[...], preferred_element_type=jnp.float32)
    z8 = jnp.zeros((feat.shape[0], 8), jnp.float32)
    fsrc_ref[...] = jnp.concatenate([feat, el, z8], axis=1)
    edst_ref[...] = jnp.concatenate([er, z8], axis=1)


def _divide_heads(s):
    cols = []
    for h in range(HEADS):
        den = s[:, 128 + h:129 + h] + 1e-9
        cols.append(s[:, h * 16:(h + 1) * 16] / den)
    return jnp.concatenate(cols, axis=1)


def _mid_body(p_ref, w_ref, a_ref, b_ref, fsrc_ref, edst_ref):
    p = p_ref[...]
    s = p[0] + p[1]
    o = _divide_heads(s)
    hmean = jnp.zeros((o.shape[0], F), jnp.float32)
    for h in range(HEADS):
        hmean = hmean + o[:, h * 16:(h + 1) * 16]
    hmean = hmean * (1.0 / HEADS)
    helu = jnp.where(hmean > 0, hmean, jnp.exp(hmean) - 1.0)
    feat = jnp.dot(helu, w_ref[...], preferred_element_type=jnp.float32)
    el = jnp.dot(feat, a_ref[...], preferred_element_type=jnp.float32)
    er = jnp.dot(feat, b_ref[...], preferred_element_type=jnp.float32)
    z8 = jnp.zeros((feat.shape[0], 8), jnp.float32)
    fsrc_ref[...] = jnp.concatenate([feat, el, z8], axis=1)
    edst_ref[...] = jnp.concatenate([er, z8], axis=1)


def _post_body(p_ref, out_ref):
    p = p_ref[...]
    out_ref[...] = _divide_heads(p[0] + p[1])


_GRID = N // BN

_pre_call = pl.pallas_call(
    _pre_body,
    grid=(_GRID,),
    in_specs=[
        pl.BlockSpec((BN, 128), lambda i: (i, 0)),
        pl.BlockSpec((128, HEADS * F), lambda i: (0, 0)),
        pl.BlockSpec((HEADS * F, HEADS), lambda i: (0, 0)),
        pl.BlockSpec((HEADS * F, HEADS), lambda i: (0, 0)),
    ],
    out_specs=[
        pl.BlockSpec((BN, FW), lambda i: (i, 0)),
        pl.BlockSpec((BN, 16), lambda i: (i, 0)),
    ],
    out_shape=[
        jax.ShapeDtypeStruct((N, FW), jnp.float32),
        jax.ShapeDtypeStruct((N, 16), jnp.float32),
    ],
)

_mid_call = pl.pallas_call(
    _mid_body,
    grid=(_GRID,),
    in_specs=[
        pl.BlockSpec((NC, BN, FW), lambda i: (0, i, 0)),
        pl.BlockSpec((F, HEADS * F), lambda i: (0, 0)),
        pl.BlockSpec((HEADS * F, HEADS), lambda i: (0, 0)),
        pl.BlockSpec((HEADS * F, HEADS), lambda i: (0, 0)),
    ],
    out_specs=[
        pl.BlockSpec((BN, FW), lambda i: (i, 0)),
        pl.BlockSpec((BN, 16), lambda i: (i, 0)),
    ],
    out_shape=[
        jax.ShapeDtypeStruct((N, FW), jnp.float32),
        jax.ShapeDtypeStruct((N, 16), jnp.float32),
    ],
)

_post_call = pl.pallas_call(
    _post_body,
    grid=(_GRID,),
    in_specs=[pl.BlockSpec((NC, BN, FW), lambda i: (0, i, 0))],
    out_specs=pl.BlockSpec((BN, HEADS * F), lambda i: (i, 0)),
    out_shape=jax.ShapeDtypeStruct((N, HEADS * F), jnp.float32),
)


def _attn_mat(a):
    hf = HEADS * F
    rows = jnp.arange(hf)
    return jnp.zeros((hf, HEADS), jnp.float32).at[rows, rows // F].set(
        a.reshape(-1))


def kernel(x, edge_index, W1, al1, ar1, W2, al2, ar2):
    eidx = jnp.stack([edge_index[0].reshape(NW, NCH, C),
                      edge_index[1].reshape(NW, NCH, C)], axis=2)
    A1, B1 = _attn_mat(al1), _attn_mat(ar1)
    A2, B2 = _attn_mat(al2), _attn_mat(ar2)

    fsrc1, edst1 = _pre_call(x, W1, A1, B1)
    p1 = _sc_gat(fsrc1, edst1, eidx)
    fsrc2, edst2 = _mid_call(p1, W2, A2, B2)
    p2 = _sc_gat(fsrc2, edst2, eidx)
    out = _post_call(p2)
    return out.reshape(N, HEADS, F)


# speedup vs baseline: 133.6061x; 1.2792x over previous
import functools

import jax
import jax.numpy as jnp
from jax import lax
from jax.experimental import pallas as pl
from jax.experimental.pallas import tpu as pltpu
from jax.experimental.pallas import tpu_sc as plsc

N = 10000
E = 320000
HEADS = 8
F = 16
FW = 144
NC, NS = 2, 16
NW = NC * NS
EPW = E // NW
C = 100
NCH = EPW // C
N_PAD = 10240
RPT = N_PAD // NS
ZR = 80
BN = 1000


def _sc_gat_body(fsrc, edst, eidx, out,
                 eidx0, eidx1, srows0, srows1, drows0, drows1,
                 gsem0, gsem1, ssem0, ssem1, acc):
    cid = lax.axis_index("c")
    sid = lax.axis_index("s")
    wid = cid * NS + sid
    buf0 = (eidx0, srows0, drows0, gsem0, ssem0)
    buf1 = (eidx1, srows1, drows1, gsem1, ssem1)

    zero16 = jnp.zeros((16,), jnp.float32)

    def zrow(r, carry):
        for j in range(FW // 16):
            srows0[r, pl.ds(j * 16, 16)] = zero16
        return carry

    lax.fori_loop(0, ZR, zrow, 0)
    for k in range(RPT // ZR):
        pltpu.sync_copy(srows0.at[pl.ds(0, ZR)],
                        acc.at[pl.ds(sid * RPT + k * ZR, ZR)])
    plsc.subcore_barrier()

    def step(j, cur, nxt, first, last):
        ceidx, csr, cdr, cgsem, cssem = cur
        neidx, nsr, ndr, ngsem, nssem = nxt
        pltpu.make_async_copy(fsrc.at[ceidx.at[0]], csr, cgsem).wait()
        pltpu.make_async_copy(edst.at[ceidx.at[1]], cdr, cgsem).wait()
        if not first:
            pltpu.make_async_copy(nsr, acc.at[neidx.at[1]], nssem).wait()
        if not last:
            pltpu.async_copy(eidx.at[wid, j + 1], neidx, ngsem)

        def edge_body(c):
            el = csr[c, pl.ds(128, 16)]
            er = cdr[c, pl.ds(0, 16)]
            s = el + er
            w16 = jnp.exp(jnp.where(s > 0, s, 0.2 * s))
            for h in range(HEADS):
                wh = w16[h]
                csr[c, pl.ds(h * 16, 16)] = csr[c, pl.ds(h * 16, 16)] * wh
            csr[c, pl.ds(128, 16)] = w16

        PRE = 16
        plsc.parallel_loop(0, PRE, unroll=4)(edge_body)
        if not last:
            pltpu.make_async_copy(eidx.at[wid, j + 1], neidx, ngsem).wait()
            pltpu.async_copy(fsrc.at[neidx.at[0]], nsr, ngsem)
            pltpu.async_copy(edst.at[neidx.at[1]], ndr, ngsem)
        plsc.parallel_loop(PRE, C, unroll=4)(edge_body)
        pltpu.async_copy(csr, acc.at[ceidx.at[1]], cssem, add=True)

    pltpu.sync_copy(eidx.at[wid, 0], eidx0)
    pltpu.async_copy(fsrc.at[eidx0.at[0]], srows0, gsem0)
    pltpu.async_copy(edst.at[eidx0.at[1]], drows0, gsem0)

    step(0, buf0, buf1, True, False)
    step(1, buf1, buf0, False, False)

    def pair(t, carry):
        step(2 * t, buf0, buf1, False, False)
        step(2 * t + 1, buf1, buf0, False, False)
        return carry

    lax.fori_loop(1, NCH // 2 - 1, pair, 0)
    step(NCH - 2, buf0, buf1, False, False)
    step(NCH - 1, buf1, buf0, False, True)
    pltpu.make_async_copy(srows1, acc.at[eidx1.at[1]], ssem1).wait()
    plsc.subcore_barrier()

    for k in range(RPT // ZR):
        r0 = sid * RPT + k * ZR
        pltpu.sync_copy(acc.at[pl.ds(r0, ZR)], out.at[cid, pl.ds(r0, ZR)])


_sc_gat = pl.kernel(
    _sc_gat_body,
    mesh=plsc.VectorSubcoreMesh(core_axis_name="c", subcore_axis_name="s"),
    compiler_params=pltpu.CompilerParams(use_tc_tiling_on_sc=False),
    out_type=jax.ShapeDtypeStruct((NC, N_PAD, FW), jnp.float32),
    scratch_types=[
        pltpu.VMEM((2, C), jnp.int32),
        pltpu.VMEM((2, C), jnp.int32),
        pltpu.VMEM((C, FW), jnp.float32),
        pltpu.VMEM((C, FW), jnp.float32),
        pltpu.VMEM((C, 16), jnp.float32),
        pltpu.VMEM((C, 16), jnp.float32),
        pltpu.SemaphoreType.DMA,
        pltpu.SemaphoreType.DMA,
        pltpu.SemaphoreType.DMA,
        pltpu.SemaphoreType.DMA,
        pltpu.VMEM_SHARED((N_PAD, FW), jnp.float32),
    ],
)


def _pre_body(x_ref, w_ref, a_ref, b_ref, fsrc_ref, edst_ref):
    feat = jnp.dot(x_ref[...], w_ref[...], preferred_element_type=jnp.float32)
    el = jnp.dot(feat, a_ref[...], preferred_element_type=jnp.float32)
    er = jnp.dot(feat, b_ref[...], preferred_element_type=jnp.float32)
    z8 = jnp.zeros((feat.shape[0], 8), jnp.float32)
    fsrc_ref[...] = jnp.concatenate([feat, el, z8], axis=1)
    edst_ref[...] = jnp.concatenate([er, z8], axis=1)


def _divide_heads(s):
    cols = []
    for h in range(HEADS):
        den = s[:, 128 + h:129 + h] + 1e-9
        cols.append(s[:, h * 16:(h + 1) * 16] / den)
    return jnp.concatenate(cols, axis=1)


def _mid_body(p_ref, w_ref, a_ref, b_ref, fsrc_ref, edst_ref):
    p = p_ref[...]
    s = p[0] + p[1]
    o = _divide_heads(s)
    hmean = jnp.zeros((o.shape[0], F), jnp.float32)
    for h in range(HEADS):
        hmean = hmean + o[:, h * 16:(h + 1) * 16]
    hmean = hmean * (1.0 / HEADS)
    helu = jnp.where(hmean > 0, hmean, jnp.exp(hmean) - 1.0)
    feat = jnp.dot(helu, w_ref[...], preferred_element_type=jnp.float32)
    el = jnp.dot(feat, a_ref[...], preferred_element_type=jnp.float32)
    er = jnp.dot(feat, b_ref[...], preferred_element_type=jnp.float32)
    z8 = jnp.zeros((feat.shape[0], 8), jnp.float32)
    fsrc_ref[...] = jnp.concatenate([feat, el, z8], axis=1)
    edst_ref[...] = jnp.concatenate([er, z8], axis=1)


def _post_body(p_ref, out_ref):
    p = p_ref[...]
    out_ref[...] = _divide_heads(p[0] + p[1])


_GRID = N // BN

_pre_call = pl.pallas_call(
    _pre_body,
    grid=(_GRID,),
    in_specs=[
        pl.BlockSpec((BN, 128), lambda i: (i, 0)),
        pl.BlockSpec((128, HEADS * F), lambda i: (0, 0)),
        pl.BlockSpec((HEADS * F, HEADS), lambda i: (0, 0)),
        pl.BlockSpec((HEADS * F, HEADS), lambda i: (0, 0)),
    ],
    out_specs=[
        pl.BlockSpec((BN, FW), lambda i: (i, 0)),
        pl.BlockSpec((BN, 16), lambda i: (i, 0)),
    ],
    out_shape=[
        jax.ShapeDtypeStruct((N, FW), jnp.float32),
        jax.ShapeDtypeStruct((N, 16), jnp.float32),
    ],
)

_mid_call = pl.pallas_call(
    _mid_body,
    grid=(_GRID,),
    in_specs=[
        pl.BlockSpec((NC, BN, FW), lambda i: (0, i, 0)),
        pl.BlockSpec((F, HEADS * F), lambda i: (0, 0)),
        pl.BlockSpec((HEADS * F, HEADS), lambda i: (0, 0)),
        pl.BlockSpec((HEADS * F, HEADS), lambda i: (0, 0)),
    ],
    out_specs=[
        pl.BlockSpec((BN, FW), lambda i: (i, 0)),
        pl.BlockSpec((BN, 16), lambda i: (i, 0)),
    ],
    out_shape=[
        jax.ShapeDtypeStruct((N, FW), jnp.float32),
        jax.ShapeDtypeStruct((N, 16), jnp.float32),
    ],
)

_post_call = pl.pallas_call(
    _post_body,
    grid=(_GRID,),
    in_specs=[pl.BlockSpec((NC, BN, FW), lambda i: (0, i, 0))],
    out_specs=pl.BlockSpec((BN, HEADS * F), lambda i: (i, 0)),
    out_shape=jax.ShapeDtypeStruct((N, HEADS * F), jnp.float32),
)


def _attn_mat(a):
    hf = HEADS * F
    rows = jnp.arange(hf)
    return jnp.zeros((hf, HEADS), jnp.float32).at[rows, rows // F].set(
        a.reshape(-1))


def kernel(x, edge_index, W1, al1, ar1, W2, al2, ar2):
    eidx = jnp.stack([edge_index[0].reshape(NW, NCH, C),
                      edge_index[1].reshape(NW, NCH, C)], axis=2)
    A1, B1 = _attn_mat(al1), _attn_mat(ar1)
    A2, B2 = _attn_mat(al2), _attn_mat(ar2)

    fsrc1, edst1 = _pre_call(x, W1, A1, B1)
    p1 = _sc_gat(fsrc1, edst1, eidx)
    fsrc2, edst2 = _mid_call(p1, W2, A2, B2)
    p2 = _sc_gat(fsrc2, edst2, eidx)
    out = _post_call(p2)
    return out.reshape(N, HEADS, F)
